# Initial kernel scaffold; baseline (speedup 1.0000x reference)
import functools

import jax
import jax.numpy as jnp
from jax import lax
from jax.experimental import pallas as pl
from jax.experimental.pallas import tpu as pltpu
from jax.experimental.pallas import tpu_sc as plsc

N_NODES = 10000
N_EDGES = 320000
D_FEAT = 128
EMBED = 128
NUM_GRAPHS = 64

NC = 2
NS = 16
NW = NC * NS
CHUNK = 128
N_STREAMS = (N_EDGES + CHUNK - 1) // CHUNK
N_STREAMS_PAD = ((N_STREAMS + NW - 1) // NW) * NW
S_PER_W = N_STREAMS_PAD // NW
E_PAD = N_STREAMS_PAD * CHUNK

D_AUG = 144
N_PAD = 10016
ROWS_PER_TILE = N_PAD // NS


def _sc_aggregate(xt, src2d, dst2d):
    mesh = plsc.VectorSubcoreMesh(core_axis_name="c", subcore_axis_name="s")

    @functools.partial(
        pl.kernel,
        out_type=jax.ShapeDtypeStruct((NC, N_PAD, D_AUG), jnp.float32),
        mesh=mesh,
        scratch_types=[
            pltpu.VMEM((S_PER_W, CHUNK), jnp.int32),
            pltpu.VMEM((S_PER_W, CHUNK), jnp.int32),
            pltpu.VMEM((CHUNK, D_AUG), jnp.float32),
            pltpu.VMEM_SHARED((N_PAD, D_AUG), jnp.float32),
            pltpu.SemaphoreType.DMA,
        ],
    )
    def k(xt_hbm, src_hbm, dst_hbm, out_hbm, src_v, dst_v, buf_v, acc_sh, sem):
        c = lax.axis_index("c")
        s = lax.axis_index("s")
        wid = s * NC + c

        zrow = s * ROWS_PER_TILE
        for t in range(CHUNK * D_AUG // 16):
            buf_v[pl.ds((t * 16) // D_AUG, 1), pl.ds((t * 16) % D_AUG, 16)] = (
                jnp.zeros((1, 16), jnp.float32)
            )
        for kk in range(4):
            pltpu.sync_copy(buf_v, acc_sh.at[pl.ds(zrow + kk * CHUNK, CHUNK)])
        pltpu.sync_copy(
            buf_v.at[pl.ds(0, ROWS_PER_TILE - 4 * CHUNK)],
            acc_sh.at[pl.ds(zrow + 4 * CHUNK, ROWS_PER_TILE - 4 * CHUNK)],
        )
        plsc.subcore_barrier()

        base = wid * S_PER_W
        pltpu.sync_copy(src_hbm.at[pl.ds(base, S_PER_W)], src_v)
        pltpu.sync_copy(dst_hbm.at[pl.ds(base, S_PER_W)], dst_v)

        def body(j, carry):
            pltpu.async_copy(xt_hbm.at[src_v.at[j]], buf_v, sem).wait()
            pltpu.sync_copy(buf_v, acc_sh.at[dst_v.at[j]], add=True)
            return carry

        lax.fori_loop(0, S_PER_W, body, 0)
        plsc.subcore_barrier()

        pltpu.sync_copy(
            acc_sh.at[pl.ds(zrow, ROWS_PER_TILE)],
            out_hbm.at[c, pl.ds(zrow, ROWS_PER_TILE)],
        )

    return k(xt, src2d, dst2d)


def _tc_finish_body(agg_ref, batch_ref, w_ref, b_ref, out_ref):
    agg = agg_ref[0] + agg_ref[1]
    feat = agg[:N_NODES, :D_FEAT]
    deg = agg[:N_NODES, D_FEAT:D_FEAT + 1]
    lin = lax.dot_general(
        feat, w_ref[...],
        dimension_numbers=(((1,), (1,)), ((), ())),
        preferred_element_type=jnp.float32,
    )
    h = jnp.maximum(lin + deg * b_ref[...], 0.0)
    gids = lax.broadcasted_iota(jnp.int32, (N_NODES, NUM_GRAPHS), 1)
    onehot = (batch_ref[...] == gids).astype(jnp.float32)
    sums = lax.dot_general(
        onehot, h,
        dimension_numbers=(((0,), (0,)), ((), ())),
        preferred_element_type=jnp.float32,
    )
    counts = lax.dot_general(
        onehot, jnp.ones((N_NODES, 1), jnp.float32),
        dimension_numbers=(((0,), (0,)), ((), ())),
        preferred_element_type=jnp.float32,
    )
    out_ref[...] = sums / jnp.maximum(counts, 1.0)


def _tc_finish(agg2, batch2d, W, b):
    return pl.pallas_call(
        _tc_finish_body,
        out_shape=jax.ShapeDtypeStruct((NUM_GRAPHS, EMBED), jnp.float32),
    )(agg2, batch2d, W, b.reshape(1, EMBED))


@jax.jit
def kernel(x, edge_index, batch, W, b):
    xt = jnp.zeros((N_PAD, D_AUG), jnp.float32)
    xt = xt.at[:N_NODES, :D_FEAT].set(x)
    xt = xt.at[:N_NODES, D_FEAT].set(1.0)

    src = edge_index[0].astype(jnp.int32)
    dst = edge_index[1].astype(jnp.int32)
    pad = E_PAD - N_EDGES
    src2d = jnp.concatenate([src, jnp.zeros((pad,), jnp.int32)]).reshape(
        N_STREAMS_PAD, CHUNK)
    dst2d = jnp.concatenate(
        [dst, jnp.full((pad,), N_NODES, jnp.int32)]).reshape(
        N_STREAMS_PAD, CHUNK)

    agg2 = _sc_aggregate(xt, src2d, dst2d)
    return _tc_finish(agg2, batch.astype(jnp.int32).reshape(N_NODES, 1), W, b)


# speedup vs baseline: 4.0928x; 4.0928x over previous
import functools

import jax
import jax.numpy as jnp
from jax import lax
from jax.experimental import pallas as pl
from jax.experimental.pallas import tpu as pltpu
from jax.experimental.pallas import tpu_sc as plsc

N_NODES = 10000
N_EDGES = 320000
D_FEAT = 128
EMBED = 128
NUM_GRAPHS = 64

NC = 2
NS = 16
NW = NC * NS
CHUNK = 128
N_STREAMS = (N_EDGES + CHUNK - 1) // CHUNK
S_PER_W = ((N_STREAMS + NW - 1) // NW + 7) // 8 * 8
N_STREAMS_PAD = S_PER_W * NW
E_PAD = N_STREAMS_PAD * CHUNK

N_PAD = 10112
ROWS_PER_TILE = N_PAD // NS


def _tc_linear_body(x_ref, w_ref, b_ref, y_ref):
    y_ref[...] = lax.dot_general(
        x_ref[...], w_ref[...],
        dimension_numbers=(((1,), (1,)), ((), ())),
        preferred_element_type=jnp.float32,
    ) + b_ref[...]


def _tc_linear(x_pad, W, b):
    return pl.pallas_call(
        _tc_linear_body,
        out_shape=jax.ShapeDtypeStruct((N_PAD, D_FEAT), jnp.float32),
    )(x_pad, W, b.reshape(1, EMBED))


def _sc_aggregate(y, src2d, dst2d, zblk):
    mesh = plsc.VectorSubcoreMesh(core_axis_name="c", subcore_axis_name="s")

    @functools.partial(
        pl.kernel,
        out_type=jax.ShapeDtypeStruct((NC, N_PAD, EMBED), jnp.float32),
        mesh=mesh,
        scratch_types=[
            pltpu.VMEM((S_PER_W, CHUNK), jnp.int32),
            pltpu.VMEM((S_PER_W, CHUNK), jnp.int32),
            pltpu.VMEM((CHUNK, EMBED), jnp.float32),
            pltpu.VMEM_SHARED((N_PAD, EMBED), jnp.float32),
            pltpu.SemaphoreType.DMA,
        ],
    )
    def k(y_hbm, src_hbm, dst_hbm, zeros_hbm, out_hbm,
          src_v, dst_v, buf_v, acc_sh, sem):
        c = lax.axis_index("c")
        s = lax.axis_index("s")
        wid = s * NC + c

        zrow = s * ROWS_PER_TILE
        pltpu.sync_copy(zeros_hbm, buf_v)
        for kk in range(4):
            pltpu.sync_copy(buf_v, acc_sh.at[pl.ds(zrow + kk * CHUNK, CHUNK)])
        pltpu.sync_copy(
            buf_v.at[pl.ds(0, ROWS_PER_TILE - 4 * CHUNK)],
            acc_sh.at[pl.ds(zrow + 4 * CHUNK, ROWS_PER_TILE - 4 * CHUNK)],
        )
        plsc.subcore_barrier()

        base = wid * S_PER_W
        pltpu.sync_copy(src_hbm.at[pl.ds(base, S_PER_W)], src_v)
        pltpu.sync_copy(dst_hbm.at[pl.ds(base, S_PER_W)], dst_v)

        def body(j, carry):
            pltpu.async_copy(y_hbm.at[src_v.at[j]], buf_v, sem).wait()
            pltpu.sync_copy(buf_v, acc_sh.at[dst_v.at[j]], add=True)
            return carry

        lax.fori_loop(0, S_PER_W, body, 0)
        plsc.subcore_barrier()

        pltpu.sync_copy(
            acc_sh.at[pl.ds(zrow, ROWS_PER_TILE)],
            out_hbm.at[c, pl.ds(zrow, ROWS_PER_TILE)],
        )

    return k(y, src2d, dst2d, zblk)


def _tc_finish_body(agg_ref, batch_ref, out_ref):
    agg = agg_ref[0] + agg_ref[1]
    h = jnp.maximum(agg[:N_NODES], 0.0)
    gids = lax.broadcasted_iota(jnp.int32, (N_NODES, NUM_GRAPHS), 1)
    onehot = (batch_ref[...] == gids).astype(jnp.float32)
    sums = lax.dot_general(
        onehot, h,
        dimension_numbers=(((0,), (0,)), ((), ())),
        preferred_element_type=jnp.float32,
    )
    counts = lax.dot_general(
        onehot, jnp.ones((N_NODES, 1), jnp.float32),
        dimension_numbers=(((0,), (0,)), ((), ())),
        preferred_element_type=jnp.float32,
    )
    out_ref[...] = sums / jnp.maximum(counts, 1.0)


def _tc_finish(agg2, batch2d):
    return pl.pallas_call(
        _tc_finish_body,
        out_shape=jax.ShapeDtypeStruct((NUM_GRAPHS, EMBED), jnp.float32),
    )(agg2, batch2d)


@jax.jit
def kernel(x, edge_index, batch, W, b):
    x_pad = jnp.zeros((N_PAD, D_FEAT), jnp.float32).at[:N_NODES].set(x)

    src = edge_index[0].astype(jnp.int32)
    dst = edge_index[1].astype(jnp.int32)
    pad = E_PAD - N_EDGES
    src2d = jnp.concatenate([src, jnp.zeros((pad,), jnp.int32)]).reshape(
        N_STREAMS_PAD, CHUNK)
    dst2d = jnp.concatenate(
        [dst, jnp.full((pad,), N_NODES, jnp.int32)]).reshape(
        N_STREAMS_PAD, CHUNK)

    y = _tc_linear(x_pad, W, b)
    agg2 = _sc_aggregate(y, src2d, dst2d,
                         jnp.zeros((CHUNK, EMBED), jnp.float32))
    return _tc_finish(agg2, batch.astype(jnp.int32).reshape(N_NODES, 1))
